# Initial kernel scaffold; baseline (speedup 1.0000x reference)
import jax
import jax.numpy as jnp
from jax.experimental import pallas as pl


def kernel(triples, graph_triples, W1, W2, relations):
    raise NotImplementedError("write your pallas kernel here")


# speedup vs baseline: 3.1351x; 3.1351x over previous
import functools

import jax
import jax.numpy as jnp
from jax import lax
from jax.experimental import pallas as pl
from jax.experimental.pallas import tpu as pltpu
from jax.experimental.pallas import tpu_sc as plsc

N = 10000
NREL = 8
NR = 2 * NREL + 1
E = 160000
D = 64
B = 50000
RN = NR * N

NC, NS = 2, 16
NW = NC * NS

RN_PAD = 171008
CSL = RN_PAD // NS

ECL = 128
ECH = 84
TPT = ECH * ECL
TP_PAD = TPT * NW
EG = 4
EOUT = ECH // EG

N_PAD = 10240
HSL = N_PAD // NS

BCH = 13
BT = BCH * ECL
B_PAD = BT * NW

_Z16 = lambda: jnp.zeros((16,), jnp.float32)


def _wid():
  return lax.axis_index("s") * NC + lax.axis_index("c")


def _zero_rows(ref, nrows):
  def body(i, _):
    for k in range(4):
      ref[i, pl.ds(k * 16, 16)] = _Z16()
    return 0
  lax.fori_loop(0, nrows, body, 0)


def _zero_flat(ref, n):
  def body(i, _):
    ref[pl.ds(i * 16, 16)] = _Z16()
    return 0
  lax.fori_loop(0, n // 16, body, 0)


@functools.cache
def _build():
  mesh = plsc.VectorSubcoreMesh(
      core_axis_name="c", subcore_axis_name="s", num_cores=NC, num_subcores=NS
  )
  sc_params = pltpu.CompilerParams(use_tc_tiling_on_sc=False, needs_layout_passes=False)

  @functools.partial(
      pl.kernel, mesh=mesh, compiler_params=sc_params,
      out_type=jax.ShapeDtypeStruct((NC * RN_PAD,), jnp.float32),
      scratch_types=[
          pltpu.VMEM((ECH, ECL), jnp.int32),
          pltpu.VMEM((ECL,), jnp.float32),
          pltpu.VMEM((CSL,), jnp.float32),
          pltpu.VMEM_SHARED((RN_PAD,), jnp.float32),
          pltpu.SemaphoreType.DMA,
      ],
  )
  def k1(seg3, out, segs_v, ones_v, zb_v, counts_sh, sem):
    core = lax.axis_index("c")
    sub = lax.axis_index("s")
    wid = _wid()
    _zero_flat(zb_v, CSL)
    for i in range(ECL // 16):
      ones_v[pl.ds(i * 16, 16)] = jnp.ones((16,), jnp.float32)
    pltpu.sync_copy(zb_v, counts_sh.at[pl.ds(sub * CSL, CSL)])
    pltpu.sync_copy(seg3.at[wid], segs_v)
    plsc.subcore_barrier()
    def hloop(oi, _):
      descs = [
          pltpu.async_copy(
              ones_v, counts_sh.at[segs_v.at[oi * 12 + k]], sem, add=True
          )
          for k in range(12)
      ]
      for d in descs:
        d.wait()
      return 0

    lax.fori_loop(0, ECH // 12, hloop, 0)
    plsc.subcore_barrier()
    pltpu.sync_copy(counts_sh.at[pl.ds(sub * CSL, CSL)], zb_v)
    pltpu.sync_copy(zb_v, out.at[pl.ds(core * RN_PAD + sub * CSL, CSL)])

  def _cn_body(c_ref, o_ref):
    c = c_ref[0] + c_ref[1]
    o_ref[...] = 1.0 / jnp.maximum(c, 1.0)

  k2 = pl.pallas_call(
      _cn_body,
      out_shape=jax.ShapeDtypeStruct((RN_PAD // 128, 128), jnp.float32),
  )

  def _layer(table_rows, gather_norm):
    out_type = [jax.ShapeDtypeStruct((NC, N_PAD, D), jnp.float32)]
    if gather_norm:
      out_type.append(jax.ShapeDtypeStruct((NW, ECH, ECL), jnp.float32))

    def body(*refs):
      if gather_norm:
        (gid3, o3, seg3, cn1, table, h_out, norm_out,
         gids_v, os_v, segs_v, norm_v, rows_v, h_sh, sem) = refs
      else:
        (gid3, o3, norm3, table, h_out,
         gids_v, os_v, norm_v, rows_v, h_sh, sem) = refs
      core = lax.axis_index("c")
      sub = lax.axis_index("s")
      wid = _wid()
      _zero_rows(rows_v, EG * ECL)
      hbase = sub * HSL
      pltpu.sync_copy(rows_v, h_sh.at[pl.ds(hbase, EG * ECL)])
      pltpu.sync_copy(
          rows_v.at[pl.ds(0, HSL - EG * ECL)],
          h_sh.at[pl.ds(hbase + EG * ECL, HSL - EG * ECL)],
      )
      pltpu.sync_copy(gid3.at[wid], gids_v)
      pltpu.sync_copy(o3.at[wid], os_v)
      if gather_norm:
        pltpu.sync_copy(seg3.at[wid], segs_v)
      else:
        pltpu.sync_copy(norm3.at[wid], norm_v)
      plsc.subcore_barrier()

      def outer(oi, _):
        descs = []
        for g in range(EG):
          t = oi * EG + g
          if gather_norm:
            descs.append(
                pltpu.async_copy(cn1.at[segs_v.at[t]], norm_v.at[t], sem)
            )
          descs.append(
              pltpu.async_copy(
                  table.at[gids_v.at[t]], rows_v.at[pl.ds(g * ECL, ECL)], sem
              )
          )
        for d in descs:
          d.wait()

        def scale(g, _):
          t = oi * EG + (g // (ECL // 16))
          gg = g % (ECL // 16)
          nv16 = norm_v[t, pl.ds(gg * 16, 16)]
          for ii in range(16):
            j = g * 16 + ii
            nv = nv16[ii]
            for k in range(4):
              sl = pl.ds(k * 16, 16)
              rows_v[j, sl] = rows_v[j, sl] * nv
          return 0
        lax.fori_loop(0, EG * ECL // 16, scale, 0)

        for g in range(EG):
          t = oi * EG + g
          pltpu.sync_copy(
              rows_v.at[pl.ds(g * ECL, ECL)], h_sh.at[os_v.at[t]], add=True
          )
        return 0

      lax.fori_loop(0, EOUT, outer, 0)
      plsc.subcore_barrier()
      pltpu.sync_copy(h_sh.at[pl.ds(hbase, EG * ECL)], rows_v)
      pltpu.sync_copy(rows_v, h_out.at[core, pl.ds(hbase, EG * ECL)])
      rest = HSL - EG * ECL
      pltpu.sync_copy(
          h_sh.at[pl.ds(hbase + EG * ECL, rest)], rows_v.at[pl.ds(0, rest)]
      )
      pltpu.sync_copy(
          rows_v.at[pl.ds(0, rest)],
          h_out.at[core, pl.ds(hbase + EG * ECL, rest)],
      )
      if gather_norm:
        pltpu.sync_copy(norm_v, norm_out.at[wid])

    scratch = [
        pltpu.VMEM((ECH, ECL), jnp.int32),
        pltpu.VMEM((ECH, ECL), jnp.int32),
    ]
    if gather_norm:
      scratch.append(pltpu.VMEM((ECH, ECL), jnp.int32))
    scratch += [
        pltpu.VMEM((ECH, ECL), jnp.float32),
        pltpu.VMEM((EG * ECL, D), jnp.float32),
        pltpu.VMEM_SHARED((N_PAD, D), jnp.float32),
        pltpu.SemaphoreType.DMA,
    ]
    return pl.kernel(
        body, mesh=mesh, out_type=tuple(out_type), scratch_types=scratch,
        compiler_params=sc_params,
    )

  k3 = _layer(None, True)
  k5 = _layer(None, False)

  NB, BN = 10, N // 10

  def _xw_body(h2_ref, w2_ref, o_ref):
    h = jnp.maximum(h2_ref[0] + h2_ref[1], 0.0)
    o_ref[0] = jnp.dot(h, w2_ref[0], preferred_element_type=jnp.float32)

  k4 = pl.pallas_call(
      _xw_body,
      grid=(NB, NR),
      in_specs=[
          pl.BlockSpec((2, BN, D), lambda n, r: (0, n, 0)),
          pl.BlockSpec((1, D, D), lambda n, r: (r, 0, 0)),
      ],
      out_specs=pl.BlockSpec((1, BN, D), lambda n, r: (r, n, 0)),
      out_shape=jax.ShapeDtypeStruct((NR, N, D), jnp.float32),
  )

  def _x_body(x2_ref, o_ref):
    o_ref[...] = x2_ref[0] + x2_ref[1]

  k6 = pl.pallas_call(
      _x_body,
      grid=(NB,),
      in_specs=[pl.BlockSpec((2, BN, D), lambda n: (0, n, 0))],
      out_specs=pl.BlockSpec((BN, D), lambda n: (n, 0)),
      out_shape=jax.ShapeDtypeStruct((N, D), jnp.float32),
  )

  @functools.partial(
      pl.kernel, mesh=mesh, compiler_params=sc_params,
      out_type=jax.ShapeDtypeStruct((B_PAD,), jnp.float32),
      scratch_types=[
          pltpu.VMEM((BCH, ECL), jnp.int32),
          pltpu.VMEM((BCH, ECL), jnp.int32),
          pltpu.VMEM((BCH, ECL), jnp.int32),
          pltpu.VMEM((ECL, D), jnp.float32),
          pltpu.VMEM((ECL, D), jnp.float32),
          pltpu.VMEM((NREL, D), jnp.float32),
          pltpu.VMEM((BT,), jnp.float32),
          pltpu.SemaphoreType.DMA,
          pltpu.SemaphoreType.DMA,
      ],
  )
  def k7(x, rel, ts3, tp3, to3, out,
         ts_v, tp_v, to_v, xs_v, xo_v, rel_v, sc_v, sem, sem2):
    wid = _wid()
    pltpu.sync_copy(rel, rel_v)
    pltpu.sync_copy(ts3.at[wid], ts_v)
    pltpu.sync_copy(tp3.at[wid], tp_v)
    pltpu.sync_copy(to3.at[wid], to_v)
    def chunk(t, _):
      d1 = pltpu.async_copy(x.at[ts_v.at[t]], xs_v, sem)
      d2 = pltpu.async_copy(x.at[to_v.at[t]], xo_v, sem2)
      d1.wait()
      d2.wait()

      def triple16(g, _):
        rj16 = tp_v[t, pl.ds(g * 16, 16)]
        lane = lax.iota(jnp.int32, 16)
        vec = _Z16()
        for ii in range(16):
          j = g * 16 + ii
          rj = rj16[ii]
          acc = _Z16()
          for k in range(4):
            sl = pl.ds(k * 16, 16)
            acc = acc + xs_v[j, sl] * xo_v[j, sl] * rel_v[rj, sl]
          vec = jnp.where(lane == ii, jnp.sum(acc), vec)
        sc_v[pl.ds(t * ECL + g * 16, 16)] = vec
        return 0

      lax.fori_loop(0, ECL // 16, triple16, 0)
      return 0

    lax.fori_loop(0, BCH, chunk, 0)
    pltpu.sync_copy(sc_v, out.at[pl.ds(wid * BT, BT)])

  return k1, k2, k3, k4, k5, k6, k7


@jax.jit
def kernel(triples, graph_triples, W1, W2, relations):
  k1, k2, k3, k4, k5, k6, k7 = _build()

  gs = graph_triples[:, 0]
  gp = graph_triples[:, 1]
  go = graph_triples[:, 2]
  nidx = jnp.arange(N, dtype=jnp.int32)
  s = jnp.concatenate([gs, go, nidx])
  p = jnp.concatenate([gp, gp + NREL, jnp.full((N,), 2 * NREL, jnp.int32)])
  o = jnp.concatenate([go, gs, nidx])
  gid = p * N + s
  seg = p * N + o
  tp_n = s.shape[0]
  pad = TP_PAD - tp_n
  gid3 = jnp.concatenate([gid, jnp.zeros((pad,), jnp.int32)]).reshape(
      NW, ECH, ECL)
  seg3 = jnp.concatenate([seg, jnp.full((pad,), RN, jnp.int32)]).reshape(
      NW, ECH, ECL)
  o3 = jnp.concatenate([o, jnp.full((pad,), N, jnp.int32)]).reshape(
      NW, ECH, ECL)

  counts2 = k1(seg3)
  cn = k2(counts2.reshape(NC, RN_PAD // 128, 128)).reshape(RN_PAD)
  h2, norm3 = k3(gid3, o3, seg3, cn, W1.reshape(RN, D))
  xw = k4(h2, W2)
  (x2,) = k5(gid3, o3, norm3, xw.reshape(RN, D))
  x = k6(x2)

  bpad = B_PAD - B
  zb = jnp.zeros((bpad,), jnp.int32)
  ts3 = jnp.concatenate([triples[:, 0], zb]).reshape(NW, BCH, ECL)
  tp3 = jnp.concatenate([triples[:, 1], zb]).reshape(NW, BCH, ECL)
  to3 = jnp.concatenate([triples[:, 2], zb]).reshape(NW, BCH, ECL)
  scores = k7(x, relations, ts3, tp3, to3)
  return scores[:B]


# speedup vs baseline: 3.9691x; 1.2660x over previous
import functools

import jax
import jax.numpy as jnp
from jax import lax
from jax.experimental import pallas as pl
from jax.experimental.pallas import tpu as pltpu
from jax.experimental.pallas import tpu_sc as plsc

N = 10000
NREL = 8
NR = 2 * NREL + 1
E = 160000
D = 64
B = 50000
RN = NR * N

NC, NS = 2, 16
NW = NC * NS

RN_PAD = 171008
CSL = RN_PAD // NS

ECL = 128
ECH = 84
TPT = ECH * ECL
TP_PAD = TPT * NW
EG = 4
EOUT = ECH // EG

N_PAD = 10240
HSL = N_PAD // NS

BCH = 13
BT = BCH * ECL
B_PAD = BT * NW

_Z16 = lambda: jnp.zeros((16,), jnp.float32)


def _wid():
  return lax.axis_index("s") * NC + lax.axis_index("c")


def _zero_rows(ref, nrows):
  def body(i, _):
    for k in range(4):
      ref[i, pl.ds(k * 16, 16)] = _Z16()
    return 0
  lax.fori_loop(0, nrows, body, 0)


def _zero_flat(ref, n):
  def body(i, _):
    ref[pl.ds(i * 16, 16)] = _Z16()
    return 0
  lax.fori_loop(0, n // 16, body, 0)


@functools.cache
def _build():
  mesh = plsc.VectorSubcoreMesh(
      core_axis_name="c", subcore_axis_name="s", num_cores=NC, num_subcores=NS
  )
  sc_params = pltpu.CompilerParams(use_tc_tiling_on_sc=False, needs_layout_passes=False)

  @functools.partial(
      pl.kernel, mesh=mesh, compiler_params=sc_params,
      out_type=jax.ShapeDtypeStruct((NC * RN_PAD,), jnp.float32),
      scratch_types=[
          pltpu.VMEM((ECH, ECL), jnp.int32),
          pltpu.VMEM((ECL,), jnp.float32),
          pltpu.VMEM((CSL,), jnp.float32),
          pltpu.VMEM_SHARED((RN_PAD,), jnp.float32),
          pltpu.SemaphoreType.DMA,
      ],
  )
  def k1(seg3, out, segs_v, ones_v, zb_v, counts_sh, sem):
    core = lax.axis_index("c")
    sub = lax.axis_index("s")
    wid = _wid()
    _zero_flat(zb_v, CSL)
    for i in range(ECL // 16):
      ones_v[pl.ds(i * 16, 16)] = jnp.ones((16,), jnp.float32)
    pltpu.sync_copy(zb_v, counts_sh.at[pl.ds(sub * CSL, CSL)])
    pltpu.sync_copy(seg3.at[wid], segs_v)
    plsc.subcore_barrier()
    def hloop(oi, _):
      descs = [
          pltpu.async_copy(
              ones_v, counts_sh.at[segs_v.at[oi * 12 + k]], sem, add=True
          )
          for k in range(12)
      ]
      for d in descs:
        d.wait()
      return 0

    lax.fori_loop(0, ECH // 12, hloop, 0)
    plsc.subcore_barrier()
    pltpu.sync_copy(counts_sh.at[pl.ds(sub * CSL, CSL)], zb_v)
    pltpu.sync_copy(zb_v, out.at[pl.ds(core * RN_PAD + sub * CSL, CSL)])

  def _cn_body(c_ref, o_ref):
    c = c_ref[0] + c_ref[1]
    o_ref[...] = 1.0 / jnp.maximum(c, 1.0)

  k2 = pl.pallas_call(
      _cn_body,
      out_shape=jax.ShapeDtypeStruct((RN_PAD // 128, 128), jnp.float32),
  )

  RPB = 2
  BR = RPB * ECL
  NBLK = ECH // RPB

  def _layer(table_rows, gather_norm):
    out_type = [jax.ShapeDtypeStruct((NC, N_PAD, D), jnp.float32)]
    if gather_norm:
      out_type.append(jax.ShapeDtypeStruct((NW, ECH, ECL), jnp.float32))

    def body(*refs):
      if gather_norm:
        (gid3, o3, seg3, cn1, table, h_out, norm_out,
         gids_v, os_v, segs_v, norm_v, rows_v, h_sh, semA, semB) = refs
      else:
        (gid3, o3, norm3, table, h_out,
         gids_v, os_v, norm_v, rows_v, h_sh, semA, semB) = refs
      core = lax.axis_index("c")
      sub = lax.axis_index("s")
      wid = _wid()
      _zero_rows(rows_v, EG * ECL)
      hbase = sub * HSL
      pltpu.sync_copy(rows_v, h_sh.at[pl.ds(hbase, EG * ECL)])
      pltpu.sync_copy(
          rows_v.at[pl.ds(0, HSL - EG * ECL)],
          h_sh.at[pl.ds(hbase + EG * ECL, HSL - EG * ECL)],
      )
      pltpu.sync_copy(gid3.at[wid], gids_v)
      pltpu.sync_copy(o3.at[wid], os_v)
      if gather_norm:
        pltpu.sync_copy(seg3.at[wid], segs_v)
      else:
        pltpu.sync_copy(norm3.at[wid], norm_v)
      plsc.subcore_barrier()

      def _descs(blk, slot, sem, issue):
        mk = pltpu.async_copy if issue else pltpu.make_async_copy
        ds = []
        for r in range(RPB):
          t = blk * RPB + r
          ds.append(mk(
              table.at[gids_v.at[t]],
              rows_v.at[pl.ds(slot * BR + r * ECL, ECL)], sem,
          ))
          if gather_norm:
            ds.append(mk(cn1.at[segs_v.at[t]], norm_v.at[t], sem))
        return ds

      def _step(blk, slot, sem):
        for d in _descs(blk, slot, sem, False):
          d.wait()

        def scale(g, _):
          t = blk * RPB + g // (ECL // 16)
          gg = g % (ECL // 16)
          nv16 = norm_v[t, pl.ds(gg * 16, 16)]
          for ii in range(16):
            j = slot * BR + g * 16 + ii
            nv = nv16[ii]
            for k in range(4):
              sl = pl.ds(k * 16, 16)
              rows_v[j, sl] = rows_v[j, sl] * nv
          return 0
        lax.fori_loop(0, BR // 16, scale, 0)

        for r in range(RPB):
          t = blk * RPB + r
          pltpu.sync_copy(
              rows_v.at[pl.ds(slot * BR + r * ECL, ECL)],
              h_sh.at[os_v.at[t]], add=True,
          )

        @pl.when(blk + 2 < NBLK)
        def _():
          _descs(blk + 2, slot, sem, True)

      _descs(0, 0, semA, True)
      _descs(1, 1, semB, True)

      def outer(oi, _):
        _step(oi * 2, 0, semA)
        _step(oi * 2 + 1, 1, semB)
        return 0

      lax.fori_loop(0, NBLK // 2, outer, 0)
      plsc.subcore_barrier()
      pltpu.sync_copy(h_sh.at[pl.ds(hbase, EG * ECL)], rows_v)
      pltpu.sync_copy(rows_v, h_out.at[core, pl.ds(hbase, EG * ECL)])
      rest = HSL - EG * ECL
      pltpu.sync_copy(
          h_sh.at[pl.ds(hbase + EG * ECL, rest)], rows_v.at[pl.ds(0, rest)]
      )
      pltpu.sync_copy(
          rows_v.at[pl.ds(0, rest)],
          h_out.at[core, pl.ds(hbase + EG * ECL, rest)],
      )
      if gather_norm:
        pltpu.sync_copy(norm_v, norm_out.at[wid])

    scratch = [
        pltpu.VMEM((ECH, ECL), jnp.int32),
        pltpu.VMEM((ECH, ECL), jnp.int32),
    ]
    if gather_norm:
      scratch.append(pltpu.VMEM((ECH, ECL), jnp.int32))
    scratch += [
        pltpu.VMEM((ECH, ECL), jnp.float32),
        pltpu.VMEM((EG * ECL, D), jnp.float32),
        pltpu.VMEM_SHARED((N_PAD, D), jnp.float32),
        pltpu.SemaphoreType.DMA,
        pltpu.SemaphoreType.DMA,
    ]
    return pl.kernel(
        body, mesh=mesh, out_type=tuple(out_type), scratch_types=scratch,
        compiler_params=sc_params,
    )

  k3 = _layer(None, True)
  k5 = _layer(None, False)

  NB, BN = 10, N // 10

  def _xw_body(h2_ref, w2_ref, o_ref):
    h = jnp.maximum(h2_ref[0] + h2_ref[1], 0.0)
    o_ref[0] = jnp.dot(h, w2_ref[0], preferred_element_type=jnp.float32)

  k4 = pl.pallas_call(
      _xw_body,
      grid=(NB, NR),
      in_specs=[
          pl.BlockSpec((2, BN, D), lambda n, r: (0, n, 0)),
          pl.BlockSpec((1, D, D), lambda n, r: (r, 0, 0)),
      ],
      out_specs=pl.BlockSpec((1, BN, D), lambda n, r: (r, n, 0)),
      out_shape=jax.ShapeDtypeStruct((NR, N, D), jnp.float32),
  )

  def _x_body(x2_ref, o_ref):
    o_ref[...] = x2_ref[0] + x2_ref[1]

  k6 = pl.pallas_call(
      _x_body,
      grid=(NB,),
      in_specs=[pl.BlockSpec((2, BN, D), lambda n: (0, n, 0))],
      out_specs=pl.BlockSpec((BN, D), lambda n: (n, 0)),
      out_shape=jax.ShapeDtypeStruct((N, D), jnp.float32),
  )

  @functools.partial(
      pl.kernel, mesh=mesh, compiler_params=sc_params,
      out_type=jax.ShapeDtypeStruct((B_PAD,), jnp.float32),
      scratch_types=[
          pltpu.VMEM((BCH, ECL), jnp.int32),
          pltpu.VMEM((BCH, ECL), jnp.int32),
          pltpu.VMEM((BCH, ECL), jnp.int32),
          pltpu.VMEM((ECL, D), jnp.float32),
          pltpu.VMEM((ECL, D), jnp.float32),
          pltpu.VMEM((NREL, D), jnp.float32),
          pltpu.VMEM((BT,), jnp.float32),
          pltpu.SemaphoreType.DMA,
          pltpu.SemaphoreType.DMA,
      ],
  )
  def k7(x, rel, ts3, tp3, to3, out,
         ts_v, tp_v, to_v, xs_v, xo_v, rel_v, sc_v, sem, sem2):
    wid = _wid()
    pltpu.sync_copy(rel, rel_v)
    pltpu.sync_copy(ts3.at[wid], ts_v)
    pltpu.sync_copy(tp3.at[wid], tp_v)
    pltpu.sync_copy(to3.at[wid], to_v)
    def chunk(t, _):
      d1 = pltpu.async_copy(x.at[ts_v.at[t]], xs_v, sem)
      d2 = pltpu.async_copy(x.at[to_v.at[t]], xo_v, sem2)
      d1.wait()
      d2.wait()

      def triple16(g, _):
        rj16 = tp_v[t, pl.ds(g * 16, 16)]
        lane = lax.iota(jnp.int32, 16)
        vec = _Z16()
        for ii in range(16):
          j = g * 16 + ii
          rj = rj16[ii]
          acc = _Z16()
          for k in range(4):
            sl = pl.ds(k * 16, 16)
            acc = acc + xs_v[j, sl] * xo_v[j, sl] * rel_v[rj, sl]
          vec = jnp.where(lane == ii, jnp.sum(acc), vec)
        sc_v[pl.ds(t * ECL + g * 16, 16)] = vec
        return 0

      lax.fori_loop(0, ECL // 16, triple16, 0)
      return 0

    lax.fori_loop(0, BCH, chunk, 0)
    pltpu.sync_copy(sc_v, out.at[pl.ds(wid * BT, BT)])

  return k1, k2, k3, k4, k5, k6, k7


@jax.jit
def kernel(triples, graph_triples, W1, W2, relations):
  k1, k2, k3, k4, k5, k6, k7 = _build()

  gs = graph_triples[:, 0]
  gp = graph_triples[:, 1]
  go = graph_triples[:, 2]
  nidx = jnp.arange(N, dtype=jnp.int32)
  s = jnp.concatenate([gs, go, nidx])
  p = jnp.concatenate([gp, gp + NREL, jnp.full((N,), 2 * NREL, jnp.int32)])
  o = jnp.concatenate([go, gs, nidx])
  gid = p * N + s
  seg = p * N + o
  tp_n = s.shape[0]
  pad = TP_PAD - tp_n
  gid3 = jnp.concatenate([gid, jnp.zeros((pad,), jnp.int32)]).reshape(
      NW, ECH, ECL)
  seg3 = jnp.concatenate([seg, jnp.full((pad,), RN, jnp.int32)]).reshape(
      NW, ECH, ECL)
  o3 = jnp.concatenate([o, jnp.full((pad,), N, jnp.int32)]).reshape(
      NW, ECH, ECL)

  counts2 = k1(seg3)
  cn = k2(counts2.reshape(NC, RN_PAD // 128, 128)).reshape(RN_PAD)
  h2, norm3 = k3(gid3, o3, seg3, cn, W1.reshape(RN, D))
  xw = k4(h2, W2)
  (x2,) = k5(gid3, o3, norm3, xw.reshape(RN, D))
  x = k6(x2)

  bpad = B_PAD - B
  zb = jnp.zeros((bpad,), jnp.int32)
  ts3 = jnp.concatenate([triples[:, 0], zb]).reshape(NW, BCH, ECL)
  tp3 = jnp.concatenate([triples[:, 1], zb]).reshape(NW, BCH, ECL)
  to3 = jnp.concatenate([triples[:, 2], zb]).reshape(NW, BCH, ECL)
  scores = k7(x, relations, ts3, tp3, to3)
  return scores[:B]


# speedup vs baseline: 4.2768x; 1.0775x over previous
import functools

import jax
import jax.numpy as jnp
from jax import lax
from jax.experimental import pallas as pl
from jax.experimental.pallas import tpu as pltpu
from jax.experimental.pallas import tpu_sc as plsc

N = 10000
NREL = 8
NR = 2 * NREL + 1
E = 160000
D = 64
B = 50000
RN = NR * N

NC, NS = 2, 16
NW = NC * NS

RN_PAD = 171008
CSL = RN_PAD // NS

ECL = 128
ECH = 84
TPT = ECH * ECL
TP_PAD = TPT * NW
EG = 4
EOUT = ECH // EG

N_PAD = 10240
HSL = N_PAD // NS

BCH = 13
BT = BCH * ECL
B_PAD = BT * NW

_Z16 = lambda: jnp.zeros((16,), jnp.float32)


def _wid():
  return lax.axis_index("s") * NC + lax.axis_index("c")


def _zero_rows(ref, nrows):
  def body(i, _):
    for k in range(4):
      ref[i, pl.ds(k * 16, 16)] = _Z16()
    return 0
  lax.fori_loop(0, nrows, body, 0)


def _zero_flat(ref, n):
  def body(i, _):
    ref[pl.ds(i * 16, 16)] = _Z16()
    return 0
  lax.fori_loop(0, n // 16, body, 0)


@functools.cache
def _build():
  mesh = plsc.VectorSubcoreMesh(
      core_axis_name="c", subcore_axis_name="s", num_cores=NC, num_subcores=NS
  )
  sc_params = pltpu.CompilerParams(use_tc_tiling_on_sc=False, needs_layout_passes=False)

  @functools.partial(
      pl.kernel, mesh=mesh, compiler_params=sc_params,
      out_type=jax.ShapeDtypeStruct((NC * RN_PAD,), jnp.float32),
      scratch_types=[
          pltpu.VMEM((ECH, ECL), jnp.int32),
          pltpu.VMEM((ECL,), jnp.float32),
          pltpu.VMEM((CSL,), jnp.float32),
          pltpu.VMEM_SHARED((RN_PAD,), jnp.float32),
          pltpu.SemaphoreType.DMA,
      ],
  )
  def k1(seg3, out, segs_v, ones_v, zb_v, counts_sh, sem):
    core = lax.axis_index("c")
    sub = lax.axis_index("s")
    wid = _wid()
    _zero_flat(zb_v, CSL)
    for i in range(ECL // 16):
      ones_v[pl.ds(i * 16, 16)] = jnp.ones((16,), jnp.float32)
    pltpu.sync_copy(zb_v, counts_sh.at[pl.ds(sub * CSL, CSL)])
    pltpu.sync_copy(seg3.at[wid], segs_v)
    plsc.subcore_barrier()
    def hloop(oi, _):
      descs = [
          pltpu.async_copy(
              ones_v, counts_sh.at[segs_v.at[oi * 12 + k]], sem, add=True
          )
          for k in range(12)
      ]
      for d in descs:
        d.wait()
      return 0

    lax.fori_loop(0, ECH // 12, hloop, 0)
    plsc.subcore_barrier()
    pltpu.sync_copy(counts_sh.at[pl.ds(sub * CSL, CSL)], zb_v)
    pltpu.sync_copy(zb_v, out.at[pl.ds(core * RN_PAD + sub * CSL, CSL)])

  def _cn_body(c_ref, o_ref):
    c = c_ref[0] + c_ref[1]
    o_ref[...] = 1.0 / jnp.maximum(c, 1.0)

  k2 = pl.pallas_call(
      _cn_body,
      out_shape=jax.ShapeDtypeStruct((RN_PAD // 128, 128), jnp.float32),
  )

  RPB = 2
  BR = RPB * ECL
  NBLK = ECH // RPB

  def _layer(table_rows, gather_norm):
    out_type = [jax.ShapeDtypeStruct((NC, N_PAD, D), jnp.float32)]
    if gather_norm:
      out_type.append(jax.ShapeDtypeStruct((NW, ECH, ECL), jnp.float32))

    def body(*refs):
      if gather_norm:
        (gid3, o3, seg3, cn1, table, h_out, norm_out,
         gids_v, os_v, segs_v, norm_v, rows_v, h_sh, semA, semB) = refs
      else:
        (gid3, o3, norm3, table, h_out,
         gids_v, os_v, norm_v, rows_v, h_sh, semA, semB) = refs
      core = lax.axis_index("c")
      sub = lax.axis_index("s")
      wid = _wid()
      _zero_rows(rows_v, EG * ECL)
      hbase = sub * HSL
      pltpu.sync_copy(rows_v, h_sh.at[pl.ds(hbase, EG * ECL)])
      pltpu.sync_copy(
          rows_v.at[pl.ds(0, HSL - EG * ECL)],
          h_sh.at[pl.ds(hbase + EG * ECL, HSL - EG * ECL)],
      )
      pltpu.sync_copy(gid3.at[wid], gids_v)
      pltpu.sync_copy(o3.at[wid], os_v)
      if gather_norm:
        pltpu.sync_copy(seg3.at[wid], segs_v)
      else:
        pltpu.sync_copy(norm3.at[wid], norm_v)
      plsc.subcore_barrier()

      def _descs(blk, slot, sem, issue):
        mk = pltpu.async_copy if issue else pltpu.make_async_copy
        ds = []
        for r in range(RPB):
          t = blk * RPB + r
          ds.append(mk(
              table.at[gids_v.at[t]],
              rows_v.at[pl.ds(slot * BR + r * ECL, ECL)], sem,
          ))
          if gather_norm:
            ds.append(mk(cn1.at[segs_v.at[t]], norm_v.at[t], sem))
        return ds

      def _step(blk, slot, sem):
        for d in _descs(blk, slot, sem, False):
          d.wait()

        def scale(g, _):
          t = blk * RPB + g // (ECL // 16)
          gg = g % (ECL // 16)
          nv16 = norm_v[t, pl.ds(gg * 16, 16)]
          for ii in range(16):
            j = slot * BR + g * 16 + ii
            nv = nv16[ii]
            for k in range(4):
              sl = pl.ds(k * 16, 16)
              rows_v[j, sl] = rows_v[j, sl] * nv
          return 0
        lax.fori_loop(0, BR // 16, scale, 0)

        for r in range(RPB):
          t = blk * RPB + r
          pltpu.sync_copy(
              rows_v.at[pl.ds(slot * BR + r * ECL, ECL)],
              h_sh.at[os_v.at[t]], add=True,
          )

        @pl.when(blk + 2 < NBLK)
        def _():
          _descs(blk + 2, slot, sem, True)

      _descs(0, 0, semA, True)
      _descs(1, 1, semB, True)

      def outer(oi, _):
        _step(oi * 2, 0, semA)
        _step(oi * 2 + 1, 1, semB)
        return 0

      lax.fori_loop(0, NBLK // 2, outer, 0)
      plsc.subcore_barrier()
      pltpu.sync_copy(h_sh.at[pl.ds(hbase, EG * ECL)], rows_v)
      pltpu.sync_copy(rows_v, h_out.at[core, pl.ds(hbase, EG * ECL)])
      rest = HSL - EG * ECL
      pltpu.sync_copy(
          h_sh.at[pl.ds(hbase + EG * ECL, rest)], rows_v.at[pl.ds(0, rest)]
      )
      pltpu.sync_copy(
          rows_v.at[pl.ds(0, rest)],
          h_out.at[core, pl.ds(hbase + EG * ECL, rest)],
      )
      if gather_norm:
        pltpu.sync_copy(norm_v, norm_out.at[wid])

    scratch = [
        pltpu.VMEM((ECH, ECL), jnp.int32),
        pltpu.VMEM((ECH, ECL), jnp.int32),
    ]
    if gather_norm:
      scratch.append(pltpu.VMEM((ECH, ECL), jnp.int32))
    scratch += [
        pltpu.VMEM((ECH, ECL), jnp.float32),
        pltpu.VMEM((EG * ECL, D), jnp.float32),
        pltpu.VMEM_SHARED((N_PAD, D), jnp.float32),
        pltpu.SemaphoreType.DMA,
        pltpu.SemaphoreType.DMA,
    ]
    return pl.kernel(
        body, mesh=mesh, out_type=tuple(out_type), scratch_types=scratch,
        compiler_params=sc_params,
    )

  k3 = _layer(None, True)
  k5 = _layer(None, False)

  NB, BN = 10, N // 10

  def _xw_body(h2_ref, w2_ref, o_ref):
    h = jnp.maximum(h2_ref[0] + h2_ref[1], 0.0)
    for r in range(NR):
      o_ref[r] = jnp.dot(h, w2_ref[r], preferred_element_type=jnp.float32)

  k4 = pl.pallas_call(
      _xw_body,
      grid=(NB,),
      in_specs=[
          pl.BlockSpec((2, BN, D), lambda n: (0, n, 0)),
          pl.BlockSpec((NR, D, D), lambda n: (0, 0, 0)),
      ],
      out_specs=pl.BlockSpec((NR, BN, D), lambda n: (0, n, 0)),
      out_shape=jax.ShapeDtypeStruct((NR, N, D), jnp.float32),
  )

  def _x_body(x2_ref, o_ref):
    o_ref[...] = x2_ref[0] + x2_ref[1]

  k6 = pl.pallas_call(
      _x_body,
      grid=(NB,),
      in_specs=[pl.BlockSpec((2, BN, D), lambda n: (0, n, 0))],
      out_specs=pl.BlockSpec((BN, D), lambda n: (n, 0)),
      out_shape=jax.ShapeDtypeStruct((N, D), jnp.float32),
  )

  @functools.partial(
      pl.kernel, mesh=mesh, compiler_params=sc_params,
      out_type=jax.ShapeDtypeStruct((B_PAD,), jnp.float32),
      scratch_types=[
          pltpu.VMEM((BCH, ECL), jnp.int32),
          pltpu.VMEM((BCH, ECL), jnp.int32),
          pltpu.VMEM((BCH, ECL), jnp.int32),
          pltpu.VMEM((ECL, D), jnp.float32),
          pltpu.VMEM((ECL, D), jnp.float32),
          pltpu.VMEM((NREL, D), jnp.float32),
          pltpu.VMEM((BT,), jnp.float32),
          pltpu.SemaphoreType.DMA,
          pltpu.SemaphoreType.DMA,
      ],
  )
  def k7(x, rel, ts3, tp3, to3, out,
         ts_v, tp_v, to_v, xs_v, xo_v, rel_v, sc_v, sem, sem2):
    wid = _wid()
    pltpu.sync_copy(rel, rel_v)
    pltpu.sync_copy(ts3.at[wid], ts_v)
    pltpu.sync_copy(tp3.at[wid], tp_v)
    pltpu.sync_copy(to3.at[wid], to_v)
    def chunk(t, _):
      d1 = pltpu.async_copy(x.at[ts_v.at[t]], xs_v, sem)
      d2 = pltpu.async_copy(x.at[to_v.at[t]], xo_v, sem2)
      d1.wait()
      d2.wait()

      def triple16(g, _):
        rj16 = tp_v[t, pl.ds(g * 16, 16)]
        lane = lax.iota(jnp.int32, 16)
        vec = _Z16()
        for ii in range(16):
          j = g * 16 + ii
          rj = rj16[ii]
          acc = _Z16()
          for k in range(4):
            sl = pl.ds(k * 16, 16)
            acc = acc + xs_v[j, sl] * xo_v[j, sl] * rel_v[rj, sl]
          vec = jnp.where(lane == ii, jnp.sum(acc), vec)
        sc_v[pl.ds(t * ECL + g * 16, 16)] = vec
        return 0

      lax.fori_loop(0, ECL // 16, triple16, 0)
      return 0

    lax.fori_loop(0, BCH, chunk, 0)
    pltpu.sync_copy(sc_v, out.at[pl.ds(wid * BT, BT)])

  return k1, k2, k3, k4, k5, k6, k7


@jax.jit
def kernel(triples, graph_triples, W1, W2, relations):
  k1, k2, k3, k4, k5, k6, k7 = _build()

  gs = graph_triples[:, 0]
  gp = graph_triples[:, 1]
  go = graph_triples[:, 2]
  nidx = jnp.arange(N, dtype=jnp.int32)
  s = jnp.concatenate([gs, go, nidx])
  p = jnp.concatenate([gp, gp + NREL, jnp.full((N,), 2 * NREL, jnp.int32)])
  o = jnp.concatenate([go, gs, nidx])
  gid = p * N + s
  seg = p * N + o
  tp_n = s.shape[0]
  pad = TP_PAD - tp_n
  gid3 = jnp.concatenate([gid, jnp.zeros((pad,), jnp.int32)]).reshape(
      NW, ECH, ECL)
  seg3 = jnp.concatenate([seg, jnp.full((pad,), RN, jnp.int32)]).reshape(
      NW, ECH, ECL)
  o3 = jnp.concatenate([o, jnp.full((pad,), N, jnp.int32)]).reshape(
      NW, ECH, ECL)

  counts2 = k1(seg3)
  cn = k2(counts2.reshape(NC, RN_PAD // 128, 128)).reshape(RN_PAD)
  h2, norm3 = k3(gid3, o3, seg3, cn, W1.reshape(RN, D))
  xw = k4(h2, W2)
  (x2,) = k5(gid3, o3, norm3, xw.reshape(RN, D))
  x = k6(x2)

  bpad = B_PAD - B
  zb = jnp.zeros((bpad,), jnp.int32)
  ts3 = jnp.concatenate([triples[:, 0], zb]).reshape(NW, BCH, ECL)
  tp3 = jnp.concatenate([triples[:, 1], zb]).reshape(NW, BCH, ECL)
  to3 = jnp.concatenate([triples[:, 2], zb]).reshape(NW, BCH, ECL)
  scores = k7(x, relations, ts3, tp3, to3)
  return scores[:B]


# speedup vs baseline: 4.2952x; 1.0043x over previous
import functools

import jax
import jax.numpy as jnp
from jax import lax
from jax.experimental import pallas as pl
from jax.experimental.pallas import tpu as pltpu
from jax.experimental.pallas import tpu_sc as plsc

N = 10000
NREL = 8
NR = 2 * NREL + 1
E = 160000
D = 64
B = 50000
RN = NR * N

NC, NS = 2, 16
NW = NC * NS

RN_PAD = 171008
CSL = RN_PAD // NS

ECL = 128
ECH = 84
TPT = ECH * ECL
TP_PAD = TPT * NW
EG = 4
EOUT = ECH // EG

N_PAD = 10240
HSL = N_PAD // NS

BCH = 13
BT = BCH * ECL
B_PAD = BT * NW

_Z16 = lambda: jnp.zeros((16,), jnp.float32)


def _wid():
  return lax.axis_index("s") * NC + lax.axis_index("c")


def _zero_rows(ref, nrows):
  def body(i, _):
    for k in range(4):
      ref[i, pl.ds(k * 16, 16)] = _Z16()
    return 0
  lax.fori_loop(0, nrows, body, 0)


def _zero_flat(ref, n):
  def body(i, _):
    ref[pl.ds(i * 16, 16)] = _Z16()
    return 0
  lax.fori_loop(0, n // 16, body, 0)


@functools.cache
def _build():
  mesh = plsc.VectorSubcoreMesh(
      core_axis_name="c", subcore_axis_name="s", num_cores=NC, num_subcores=NS
  )
  sc_params = pltpu.CompilerParams(use_tc_tiling_on_sc=False, needs_layout_passes=False)

  @functools.partial(
      pl.kernel, mesh=mesh, compiler_params=sc_params,
      out_type=jax.ShapeDtypeStruct((NC * RN_PAD,), jnp.float32),
      scratch_types=[
          pltpu.VMEM((ECH, ECL), jnp.int32),
          pltpu.VMEM((ECL,), jnp.float32),
          pltpu.VMEM((CSL,), jnp.float32),
          pltpu.VMEM_SHARED((RN_PAD,), jnp.float32),
          pltpu.SemaphoreType.DMA,
      ],
  )
  def k1(seg3, out, segs_v, ones_v, zb_v, counts_sh, sem):
    core = lax.axis_index("c")
    sub = lax.axis_index("s")
    wid = _wid()
    _zero_flat(zb_v, CSL)
    for i in range(ECL // 16):
      ones_v[pl.ds(i * 16, 16)] = jnp.ones((16,), jnp.float32)
    pltpu.sync_copy(zb_v, counts_sh.at[pl.ds(sub * CSL, CSL)])
    pltpu.sync_copy(seg3.at[wid], segs_v)
    plsc.subcore_barrier()
    def hloop(oi, _):
      descs = [
          pltpu.async_copy(
              ones_v, counts_sh.at[segs_v.at[oi * 12 + k]], sem, add=True
          )
          for k in range(12)
      ]
      for d in descs:
        d.wait()
      return 0

    lax.fori_loop(0, ECH // 12, hloop, 0)
    plsc.subcore_barrier()
    pltpu.sync_copy(counts_sh.at[pl.ds(sub * CSL, CSL)], zb_v)
    pltpu.sync_copy(zb_v, out.at[pl.ds(core * RN_PAD + sub * CSL, CSL)])

  def _cn_body(c_ref, o_ref):
    c = c_ref[0] + c_ref[1]
    o_ref[...] = 1.0 / jnp.maximum(c, 1.0)

  k2 = pl.pallas_call(
      _cn_body,
      out_shape=jax.ShapeDtypeStruct((RN_PAD // 128, 128), jnp.float32),
  )

  RPB = 1
  BR = RPB * ECL
  NBLK = ECH // RPB

  def _layer(table_rows, gather_norm):
    out_type = [jax.ShapeDtypeStruct((NC, N_PAD, D), jnp.float32)]
    if gather_norm:
      out_type.append(jax.ShapeDtypeStruct((NW, ECH, ECL), jnp.float32))

    def body(*refs):
      if gather_norm:
        (gid3, o3, seg3, cn1, table, h_out, norm_out,
         gids_v, os_v, segs_v, norm_v, rows_v, h_sh, *sems) = refs
      else:
        (gid3, o3, norm3, table, h_out,
         gids_v, os_v, norm_v, rows_v, h_sh, *sems) = refs
      gsem = sems[:3]
      ssem = sems[3:]
      core = lax.axis_index("c")
      sub = lax.axis_index("s")
      wid = _wid()
      SR = 3 * BR
      _zero_rows(rows_v, SR)
      hbase = sub * HSL
      pltpu.sync_copy(rows_v, h_sh.at[pl.ds(hbase, SR)])
      pltpu.sync_copy(
          rows_v.at[pl.ds(0, HSL - SR)],
          h_sh.at[pl.ds(hbase + SR, HSL - SR)],
      )
      pltpu.sync_copy(gid3.at[wid], gids_v)
      pltpu.sync_copy(o3.at[wid], os_v)
      if gather_norm:
        pltpu.sync_copy(seg3.at[wid], segs_v)
      else:
        pltpu.sync_copy(norm3.at[wid], norm_v)
      plsc.subcore_barrier()

      def _gdescs(blk, slot, issue):
        mk = pltpu.async_copy if issue else pltpu.make_async_copy
        ds = []
        for r in range(RPB):
          t = blk * RPB + r
          ds.append(mk(
              table.at[gids_v.at[t]],
              rows_v.at[pl.ds(slot * BR + r * ECL, ECL)], gsem[slot],
          ))
          if gather_norm:
            ds.append(mk(cn1.at[segs_v.at[t]], norm_v.at[t], gsem[slot]))
        return ds

      def _sdescs(blk, slot, issue):
        ds = []
        for r in range(RPB):
          t = blk * RPB + r
          dst = rows_v.at[pl.ds(slot * BR + r * ECL, ECL)]
          if issue:
            ds.append(pltpu.async_copy(
                dst, h_sh.at[os_v.at[t]], ssem[slot], add=True))
          else:
            ds.append(pltpu.make_async_copy(
                table.at[pl.ds(0, ECL)], dst, ssem[slot]))
        return ds

      def _step(blk, slot, pslot):
        for d in _gdescs(blk, slot, False):
          d.wait()

        def scale(g, _):
          t = blk * RPB + g // (ECL // 16)
          gg = g % (ECL // 16)
          nv16 = norm_v[t, pl.ds(gg * 16, 16)]
          for ii in range(16):
            j = slot * BR + g * 16 + ii
            nv = nv16[ii]
            for k in range(4):
              sl = pl.ds(k * 16, 16)
              rows_v[j, sl] = rows_v[j, sl] * nv
          return 0
        lax.fori_loop(0, BR // 16, scale, 0)

        _sdescs(blk, slot, True)

        @pl.when(blk >= 1)
        def _():
          for d in _sdescs(blk - 1, pslot, False):
            d.wait()

        @pl.when(blk + 2 < NBLK)
        def _():
          _gdescs(blk + 2, pslot, True)

      _gdescs(0, 0, True)
      _gdescs(1, 1, True)

      def outer(oi, _):
        for j in range(3):
          _step(oi * 3 + j, j, (j + 2) % 3)
        return 0

      lax.fori_loop(0, NBLK // 3, outer, 0)
      for d in _sdescs(NBLK - 1, (NBLK - 1) % 3, False):
        d.wait()
      plsc.subcore_barrier()
      pltpu.sync_copy(h_sh.at[pl.ds(hbase, SR)], rows_v)
      pltpu.sync_copy(rows_v, h_out.at[core, pl.ds(hbase, SR)])
      rest = HSL - SR
      pltpu.sync_copy(
          h_sh.at[pl.ds(hbase + SR, rest)], rows_v.at[pl.ds(0, rest)]
      )
      pltpu.sync_copy(
          rows_v.at[pl.ds(0, rest)],
          h_out.at[core, pl.ds(hbase + SR, rest)],
      )
      if gather_norm:
        pltpu.sync_copy(norm_v, norm_out.at[wid])

    scratch = [
        pltpu.VMEM((ECH, ECL), jnp.int32),
        pltpu.VMEM((ECH, ECL), jnp.int32),
    ]
    if gather_norm:
      scratch.append(pltpu.VMEM((ECH, ECL), jnp.int32))
    scratch += [
        pltpu.VMEM((ECH, ECL), jnp.float32),
        pltpu.VMEM((3 * BR, D), jnp.float32),
        pltpu.VMEM_SHARED((N_PAD, D), jnp.float32),
    ] + [pltpu.SemaphoreType.DMA] * 6
    return pl.kernel(
        body, mesh=mesh, out_type=tuple(out_type), scratch_types=scratch,
        compiler_params=sc_params,
    )

  k3 = _layer(None, True)
  k5 = _layer(None, False)

  NB, BN = 10, N // 10

  def _xw_body(h2_ref, w2_ref, o_ref):
    h = jnp.maximum(h2_ref[0] + h2_ref[1], 0.0)
    for r in range(NR):
      o_ref[r] = jnp.dot(h, w2_ref[r], preferred_element_type=jnp.float32)

  k4 = pl.pallas_call(
      _xw_body,
      grid=(NB,),
      in_specs=[
          pl.BlockSpec((2, BN, D), lambda n: (0, n, 0)),
          pl.BlockSpec((NR, D, D), lambda n: (0, 0, 0)),
      ],
      out_specs=pl.BlockSpec((NR, BN, D), lambda n: (0, n, 0)),
      out_shape=jax.ShapeDtypeStruct((NR, N, D), jnp.float32),
  )

  def _x_body(x2_ref, o_ref):
    o_ref[...] = x2_ref[0] + x2_ref[1]

  k6 = pl.pallas_call(
      _x_body,
      grid=(NB,),
      in_specs=[pl.BlockSpec((2, BN, D), lambda n: (0, n, 0))],
      out_specs=pl.BlockSpec((BN, D), lambda n: (n, 0)),
      out_shape=jax.ShapeDtypeStruct((N, D), jnp.float32),
  )

  @functools.partial(
      pl.kernel, mesh=mesh, compiler_params=sc_params,
      out_type=jax.ShapeDtypeStruct((B_PAD,), jnp.float32),
      scratch_types=[
          pltpu.VMEM((BCH, ECL), jnp.int32),
          pltpu.VMEM((BCH, ECL), jnp.int32),
          pltpu.VMEM((BCH, ECL), jnp.int32),
          pltpu.VMEM((ECL, D), jnp.float32),
          pltpu.VMEM((ECL, D), jnp.float32),
          pltpu.VMEM((NREL, D), jnp.float32),
          pltpu.VMEM((BT,), jnp.float32),
          pltpu.SemaphoreType.DMA,
          pltpu.SemaphoreType.DMA,
      ],
  )
  def k7(x, rel, ts3, tp3, to3, out,
         ts_v, tp_v, to_v, xs_v, xo_v, rel_v, sc_v, sem, sem2):
    wid = _wid()
    pltpu.sync_copy(rel, rel_v)
    pltpu.sync_copy(ts3.at[wid], ts_v)
    pltpu.sync_copy(tp3.at[wid], tp_v)
    pltpu.sync_copy(to3.at[wid], to_v)
    def chunk(t, _):
      d1 = pltpu.async_copy(x.at[ts_v.at[t]], xs_v, sem)
      d2 = pltpu.async_copy(x.at[to_v.at[t]], xo_v, sem2)
      d1.wait()
      d2.wait()

      def triple16(g, _):
        rj16 = tp_v[t, pl.ds(g * 16, 16)]
        lane = lax.iota(jnp.int32, 16)
        vec = _Z16()
        for ii in range(16):
          j = g * 16 + ii
          rj = rj16[ii]
          acc = _Z16()
          for k in range(4):
            sl = pl.ds(k * 16, 16)
            acc = acc + xs_v[j, sl] * xo_v[j, sl] * rel_v[rj, sl]
          vec = jnp.where(lane == ii, jnp.sum(acc), vec)
        sc_v[pl.ds(t * ECL + g * 16, 16)] = vec
        return 0

      lax.fori_loop(0, ECL // 16, triple16, 0)
      return 0

    lax.fori_loop(0, BCH, chunk, 0)
    pltpu.sync_copy(sc_v, out.at[pl.ds(wid * BT, BT)])

  return k1, k2, k3, k4, k5, k6, k7


@jax.jit
def kernel(triples, graph_triples, W1, W2, relations):
  k1, k2, k3, k4, k5, k6, k7 = _build()

  gs = graph_triples[:, 0]
  gp = graph_triples[:, 1]
  go = graph_triples[:, 2]
  nidx = jnp.arange(N, dtype=jnp.int32)
  s = jnp.concatenate([gs, go, nidx])
  p = jnp.concatenate([gp, gp + NREL, jnp.full((N,), 2 * NREL, jnp.int32)])
  o = jnp.concatenate([go, gs, nidx])
  gid = p * N + s
  seg = p * N + o
  tp_n = s.shape[0]
  pad = TP_PAD - tp_n
  gid3 = jnp.concatenate([gid, jnp.zeros((pad,), jnp.int32)]).reshape(
      NW, ECH, ECL)
  seg3 = jnp.concatenate([seg, jnp.full((pad,), RN, jnp.int32)]).reshape(
      NW, ECH, ECL)
  o3 = jnp.concatenate([o, jnp.full((pad,), N, jnp.int32)]).reshape(
      NW, ECH, ECL)

  counts2 = k1(seg3)
  cn = k2(counts2.reshape(NC, RN_PAD // 128, 128)).reshape(RN_PAD)
  h2, norm3 = k3(gid3, o3, seg3, cn, W1.reshape(RN, D))
  xw = k4(h2, W2)
  (x2,) = k5(gid3, o3, norm3, xw.reshape(RN, D))
  x = k6(x2)

  bpad = B_PAD - B
  zb = jnp.zeros((bpad,), jnp.int32)
  ts3 = jnp.concatenate([triples[:, 0], zb]).reshape(NW, BCH, ECL)
  tp3 = jnp.concatenate([triples[:, 1], zb]).reshape(NW, BCH, ECL)
  to3 = jnp.concatenate([triples[:, 2], zb]).reshape(NW, BCH, ECL)
  scores = k7(x, relations, ts3, tp3, to3)
  return scores[:B]
